# Initial kernel scaffold; baseline (speedup 1.0000x reference)
import jax
import jax.numpy as jnp
from jax.experimental import pallas as pl


def kernel(x, edge_index, edge_weight, W):
    raise NotImplementedError("write your pallas kernel here")


# speedup vs baseline: 6.3446x; 6.3446x over previous
import functools

import jax
import jax.numpy as jnp
from jax import lax
from jax.experimental import pallas as pl
from jax.experimental.pallas import tpu as pltpu
from jax.experimental.pallas import tpu_sc as plsc

NC = 2
NS = 16
NW = NC * NS
CHUNK = 80
BLK = 25
LANES = 16


def _matmul(x, W):
    n, d_in = x.shape
    d_out = W.shape[1]
    bm = 1000
    grid = (n // bm,)

    def body(x_ref, w_ref, o_ref):
        o_ref[...] = jnp.dot(x_ref[...], w_ref[...],
                             preferred_element_type=jnp.float32)

    return pl.pallas_call(
        body,
        grid=grid,
        in_specs=[
            pl.BlockSpec((bm, d_in), lambda i: (i, 0)),
            pl.BlockSpec((d_in, d_out), lambda i: (0, 0)),
        ],
        out_specs=pl.BlockSpec((bm, d_out), lambda i: (i, 0)),
        out_shape=jax.ShapeDtypeStruct((n, d_out), jnp.float32),
    )(x, W)


def _combine(partials):
    _, n, d = partials.shape
    bm = 1000
    grid = (n // bm,)

    def body(p_ref, o_ref):
        o_ref[...] = p_ref[0] + p_ref[1]

    return pl.pallas_call(
        body,
        grid=grid,
        in_specs=[pl.BlockSpec((2, bm, d), lambda i: (0, i, 0))],
        out_specs=pl.BlockSpec((bm, d), lambda i: (i, 0)),
        out_shape=jax.ShapeDtypeStruct((n, d), jnp.float32),
    )(partials)


def _sc_spmm(support, src3, dst3, ew3, zeros):
    n, d = support.shape
    nblk = src3.shape[1]
    rows_per_sub = (n // (8 * NS)) * 8
    n_rem = n - NS * rows_per_sub
    d_regs = d // LANES

    mesh = plsc.VectorSubcoreMesh(core_axis_name="c", subcore_axis_name="s")

    @functools.partial(
        pl.kernel,
        out_type=jax.ShapeDtypeStruct((NC, n, d), jnp.float32),
        mesh=mesh,
        scratch_types=[
            pltpu.VMEM((BLK, CHUNK), jnp.int32),
            pltpu.VMEM((BLK, CHUNK), jnp.int32),
            pltpu.VMEM((BLK * CHUNK,), jnp.float32),
            pltpu.VMEM((CHUNK, d), jnp.float32),
            pltpu.VMEM_SHARED((n, d), jnp.float32),
            pltpu.SemaphoreType.DMA,
        ],
    )
    def k(support_hbm, src_hbm, dst_hbm, ew_hbm, zeros_hbm, out_hbm,
          src_v, dst_v, ew_v, rows_v, acc, sem):
        c = lax.axis_index("c")
        s = lax.axis_index("s")
        wid = s * NC + c

        row0 = s * rows_per_sub
        pltpu.sync_copy(zeros_hbm.at[pl.ds(row0, rows_per_sub)],
                        acc.at[pl.ds(row0, rows_per_sub)])
        if n_rem:
            @pl.when(s == 0)
            def _():
                pltpu.sync_copy(zeros_hbm.at[pl.ds(NS * rows_per_sub, n_rem)],
                                acc.at[pl.ds(NS * rows_per_sub, n_rem)])

        plsc.subcore_barrier()

        def block_body(b, _):
            pltpu.sync_copy(src_hbm.at[wid, b], src_v)
            pltpu.sync_copy(dst_hbm.at[wid, b], dst_v)
            pltpu.sync_copy(ew_hbm.at[wid, b], ew_v)

            def chunk_body(ci, _):
                pltpu.async_copy(support_hbm.at[src_v.at[ci]], rows_v,
                                 sem).wait()

                def group_body(g, _):
                    w16 = ew_v[pl.ds(ci * CHUNK + g * LANES, LANES)]
                    for j in range(LANES):
                        e = g * LANES + j
                        w = w16[j]
                        for dd in range(d_regs):
                            sl = pl.ds(dd * LANES, LANES)
                            rows_v[e, sl] = rows_v[e, sl] * w
                    return _

                lax.fori_loop(0, CHUNK // LANES, group_body, None)

                pltpu.sync_copy(rows_v, acc.at[dst_v.at[ci]], add=True)
                return _

            lax.fori_loop(0, BLK, chunk_body, None)
            return _

        lax.fori_loop(0, nblk, block_body, None)
        plsc.subcore_barrier()

        pltpu.sync_copy(acc.at[pl.ds(row0, rows_per_sub)],
                        out_hbm.at[c, pl.ds(row0, rows_per_sub)])
        if n_rem:
            @pl.when(s == 0)
            def _():
                pltpu.sync_copy(acc.at[pl.ds(NS * rows_per_sub, n_rem)],
                                out_hbm.at[c, pl.ds(NS * rows_per_sub, n_rem)])

    return k(support, src3, dst3, ew3, zeros)


def kernel(x, edge_index, edge_weight, W):
    n, _ = x.shape
    d = W.shape[1]
    e = edge_weight.shape[0]
    epw = e // NW
    nch = epw // CHUNK

    nblk = nch // BLK

    support = _matmul(x, W)

    src3 = edge_index[0].reshape(NW, nblk, BLK, CHUNK)
    dst3 = edge_index[1].reshape(NW, nblk, BLK, CHUNK)
    ew3 = edge_weight.reshape(NW, nblk, BLK * CHUNK)
    zeros = jnp.zeros((n, d), jnp.float32)

    partials = _sc_spmm(support, src3, dst3, ew3, zeros)
    return _combine(partials)
